# baseline (speedup 1.0000x reference)
import functools

import jax
import jax.numpy as jnp
from jax import lax
from jax.experimental import pallas as pl
from jax.experimental.pallas import tpu as pltpu
from jax.experimental.pallas import tpu_sc as plsc

_B, _F, _K, _D = 16384, 100, 50, 64
_NW = 32
_ROWS_PER_W = _B // _NW
_CHUNK = 4
_NCHUNK = _ROWS_PER_W // _CHUNK
_IN_C = _CHUNK * _K * 2 * _D
_OUT_C = _CHUNK * _K * _D
_GROUPS = _CHUNK * _K


def _make_sc_kernel():
    mesh = plsc.VectorSubcoreMesh(core_axis_name="c", subcore_axis_name="s")

    @functools.partial(
        pl.kernel,
        mesh=mesh,
        out_type=jax.ShapeDtypeStruct((_B * _K * _D,), jnp.float32),
        scratch_types=[
            pltpu.VMEM((2, _IN_C), jnp.float32),
            pltpu.VMEM((2, _OUT_C), jnp.float32),
            [pltpu.SemaphoreType.DMA] * 2,
            [pltpu.SemaphoreType.DMA] * 2,
        ],
        compiler_params=pltpu.CompilerParams(use_tc_tiling_on_sc=False),
    )
    def sc_copy(in_hbm, out_hbm, bin_, bout, rsem, wsem):
        wid = lax.axis_index("s") * 2 + lax.axis_index("c")
        in_base = wid * (_ROWS_PER_W * _K * 2 * _D)
        out_base = wid * (_ROWS_PER_W * _K * _D)

        def start_read(g, slot):
            pltpu.async_copy(
                in_hbm.at[pl.ds(in_base + g * _IN_C, _IN_C)], bin_.at[slot],
                rsem[slot])

        def start_write(g, slot):
            pltpu.async_copy(
                bout.at[slot], out_hbm.at[pl.ds(out_base + g * _OUT_C, _OUT_C)],
                wsem[slot])

        def wait_read(slot):
            pltpu.make_async_copy(
                in_hbm.at[pl.ds(in_base, _IN_C)], bin_.at[slot],
                rsem[slot]).wait()

        def wait_write(slot):
            pltpu.make_async_copy(
                bout.at[slot], out_hbm.at[pl.ds(out_base, _OUT_C)],
                wsem[slot]).wait()

        def compact(slot):
            @plsc.parallel_loop(0, _GROUPS, unroll=4)
            def _(i):
                src = pl.multiple_of(i * (2 * _D), 2 * _D)
                dst = pl.multiple_of(i * _D, _D)
                for j in range(4):
                    bout[slot, pl.ds(dst + j * 16, 16)] = (
                        bin_[slot, pl.ds(src + j * 16, 16)])

        start_read(0, 0)
        start_read(1, 1)
        for g in (0, 1):
            wait_read(g)
            compact(g)
            start_write(g, g)
            start_read(g + 2, g)

        @pl.loop(2, _NCHUNK - 2, step=2)
        def _(g0):
            for b in range(2):
                g = g0 + b
                wait_read(b)
                wait_write(b)
                compact(b)
                start_write(g, b)
                start_read(g + 2, b)

        for g in (_NCHUNK - 2, _NCHUNK - 1):
            slot = g % 2
            wait_read(slot)
            wait_write(slot)
            compact(slot)
            start_write(g, slot)
        for slot in (0, 1):
            wait_write(slot)

    return sc_copy


_SC_KERNEL = _make_sc_kernel()


def kernel(tensor):
    flat = tensor.reshape(_B * _K * 2 * _D)
    return _SC_KERNEL(flat).reshape(_B, _K, _D)


# speedup vs baseline: 1.2412x; 1.2412x over previous
import functools

import jax
import jax.numpy as jnp
from jax import lax
from jax.experimental import pallas as pl
from jax.experimental.pallas import tpu as pltpu
from jax.experimental.pallas import tpu_sc as plsc

_B, _F, _K, _D = 16384, 100, 50, 64
_NW = 32
_ROWS_PER_W = _B // _NW
_CHUNK = 2
_NCHUNK = _ROWS_PER_W // _CHUNK


def _make_sc_kernel():
    mesh = plsc.VectorSubcoreMesh(core_axis_name="c", subcore_axis_name="s")

    @functools.partial(
        pl.kernel,
        mesh=mesh,
        out_type=jax.ShapeDtypeStruct((_B, _K, _D), jnp.float32),
        scratch_types=[
            pltpu.VMEM((2, _CHUNK, _F, _D), jnp.float32),
            pltpu.VMEM((2, _CHUNK, _K, _D), jnp.float32),
            [pltpu.SemaphoreType.DMA] * 2,
            [pltpu.SemaphoreType.DMA] * 2,
        ],
    )
    def sc_copy(in_hbm, out_hbm, bin_, bout, rsem, wsem):
        wid = lax.axis_index("s") * 2 + lax.axis_index("c")
        base = wid * _ROWS_PER_W

        def start_read(g, slot):
            pltpu.async_copy(
                in_hbm.at[pl.ds(base + g * _CHUNK, _CHUNK)], bin_.at[slot],
                rsem[slot])

        def start_write(g, slot):
            pltpu.async_copy(
                bout.at[slot], out_hbm.at[pl.ds(base + g * _CHUNK, _CHUNK)],
                wsem[slot])

        def wait_read(slot):
            pltpu.make_async_copy(
                in_hbm.at[pl.ds(base, _CHUNK)], bin_.at[slot],
                rsem[slot]).wait()

        def wait_write(slot):
            pltpu.make_async_copy(
                bout.at[slot], out_hbm.at[pl.ds(base, _CHUNK)],
                wsem[slot]).wait()

        def compact(slot):
            for c in range(_CHUNK):
                @plsc.parallel_loop(0, _K, unroll=2)
                def _(k):
                    for j in range(4):
                        bout[slot, c, k, pl.ds(j * 16, 16)] = (
                            bin_[slot, c, 2 * k, pl.ds(j * 16, 16)])

        start_read(0, 0)
        start_read(1, 1)
        for g in (0, 1):
            wait_read(g)
            compact(g)
            start_write(g, g)
            start_read(g + 2, g)

        @pl.loop(2, _NCHUNK - 2, step=2)
        def _(g0):
            for b in range(2):
                g = g0 + b
                wait_read(b)
                wait_write(b)
                compact(b)
                start_write(g, b)
                start_read(g + 2, b)

        for g in (_NCHUNK - 2, _NCHUNK - 1):
            slot = g % 2
            wait_read(slot)
            wait_write(slot)
            compact(slot)
            start_write(g, slot)
        for slot in (0, 1):
            wait_write(slot)

    return sc_copy


_SC_KERNEL = _make_sc_kernel()


def kernel(tensor):
    return _SC_KERNEL(tensor)


# speedup vs baseline: 10.3060x; 8.3030x over previous
import functools

import jax
import jax.numpy as jnp
from jax import lax
from jax.experimental import pallas as pl
from jax.experimental.pallas import tpu as pltpu
from jax.experimental.pallas import tpu_sc as plsc

_B, _F, _K, _D = 16384, 100, 50, 64
_NW = 32
_LANES = _B // _NW
_NCHUNK = _K


def _make_sc_kernel():
    mesh = plsc.VectorSubcoreMesh(core_axis_name="c", subcore_axis_name="s")

    @functools.partial(
        pl.kernel,
        mesh=mesh,
        out_type=jax.ShapeDtypeStruct((_K, _D, _B), jnp.float32),
        scratch_types=[
            pltpu.VMEM((2, _D, _LANES), jnp.float32),
            [pltpu.SemaphoreType.DMA] * 2,
            [pltpu.SemaphoreType.DMA] * 2,
        ],
    )
    def sc_copy(in_hbm, out_hbm, buf, rsem, wsem):
        wid = lax.axis_index("s") * 2 + lax.axis_index("c")
        lane0 = wid * _LANES

        def start_read(k, slot):
            pltpu.async_copy(
                in_hbm.at[2 * k, :, pl.ds(lane0, _LANES)], buf.at[slot],
                rsem[slot])

        def start_write(k, slot):
            pltpu.async_copy(
                buf.at[slot], out_hbm.at[k, :, pl.ds(lane0, _LANES)],
                wsem[slot])

        def wait_read(slot):
            pltpu.make_async_copy(
                in_hbm.at[0, :, pl.ds(lane0, _LANES)], buf.at[slot],
                rsem[slot]).wait()

        def wait_write(slot):
            pltpu.make_async_copy(
                buf.at[slot], out_hbm.at[0, :, pl.ds(lane0, _LANES)],
                wsem[slot]).wait()

        start_read(0, 0)
        start_read(1, 1)
        for k in (0, 1):
            wait_read(k)
            start_write(k, k)
            start_read(k + 2, k)

        @pl.loop(2, _NCHUNK - 2, step=2)
        def _(k0):
            for b in range(2):
                k = k0 + b
                wait_read(b)
                wait_write(b)
                start_write(k, b)
                start_read(k + 2, b)

        for k in (_NCHUNK - 2, _NCHUNK - 1):
            slot = k % 2
            wait_read(slot)
            wait_write(slot)
            start_write(k, slot)
        for slot in (0, 1):
            wait_write(slot)

    return sc_copy


_SC_KERNEL = _make_sc_kernel()


def kernel(tensor):
    x_t = jnp.transpose(tensor, (1, 2, 0))
    out_t = _SC_KERNEL(x_t)
    return jnp.transpose(out_t, (2, 0, 1))
